# Initial kernel scaffold; baseline (speedup 1.0000x reference)
import jax
import jax.numpy as jnp
from jax.experimental import pallas as pl


def kernel(place_features, transition_features, pre_edge_index, post_edge_index, W_ptm, b_ptm, W_tpm, b_tpm, W_pu, b_pu, W_tu, b_tu, W_pa, b_pa, W_ta, b_ta):
    raise NotImplementedError("write your pallas kernel here")


# speedup vs baseline: 3.0729x; 3.0729x over previous
import functools

import jax
import jax.numpy as jnp
from jax import lax
from jax.experimental import pallas as pl
from jax.experimental.pallas import tpu as pltpu
from jax.experimental.pallas import tpu_sc as plsc

N = 10000
D = 256
HH = 256
HC = 128
RB = 2000
K = 80
NC = 2
NS = 16
NL = 16
NPAD = 10240


def _transform_body(x_ref, w_ref, b_ref, wa_ref, ba_ref,
                    xw_ref, a_ref, m_ref, msc):
    i = pl.program_id(0)
    xw = jnp.dot(x_ref[...], w_ref[...],
                 preferred_element_type=jnp.float32) + b_ref[...]
    xw_ref[...] = xw
    a = jnp.dot(xw, wa_ref[...],
                preferred_element_type=jnp.float32) + ba_ref[...]
    a_ref[...] = a
    bm = jnp.max(a)

    @pl.when(i == 0)
    def _():
        msc[0, 0] = bm

    @pl.when(i > 0)
    def _():
        msc[0, 0] = jnp.maximum(msc[0, 0], bm)

    m_ref[...] = jnp.full((1, 1), msc[0, 0], jnp.float32)


def _node_transform(x, w, b, wa, ba):
    return pl.pallas_call(
        _transform_body,
        grid=(N // RB,),
        in_specs=[
            pl.BlockSpec((RB, D), lambda i: (i, 0)),
            pl.BlockSpec((D, HH), lambda i: (0, 0)),
            pl.BlockSpec((1, HH), lambda i: (0, 0)),
            pl.BlockSpec((D, 1), lambda i: (0, 0)),
            pl.BlockSpec((1, 1), lambda i: (0, 0)),
        ],
        out_specs=[
            pl.BlockSpec((RB, HH), lambda i: (i, 0)),
            pl.BlockSpec((RB, 1), lambda i: (i, 0)),
            pl.BlockSpec((1, 1), lambda i: (0, 0)),
        ],
        out_shape=[
            jax.ShapeDtypeStruct((N, HH), jnp.float32),
            jax.ShapeDtypeStruct((N, 1), jnp.float32),
            jax.ShapeDtypeStruct((1, 1), jnp.float32),
        ],
        scratch_shapes=[pltpu.SMEM((1, 1), jnp.float32)],
    )(x, w, b.reshape(1, HH), wa, ba.reshape(1, 1))


def _weight_body(xw_ref, a_ref, m_ref, y_ref, w_ref):
    w = jnp.exp(a_ref[...] - m_ref[0, 0])
    w_ref[...] = w
    y = xw_ref[...] * w
    y_ref[0, :, :] = y[:, :HC]
    y_ref[1, :, :] = y[:, HC:]


def _node_weight(xw, a, m):
    return pl.pallas_call(
        _weight_body,
        grid=(N // RB,),
        in_specs=[
            pl.BlockSpec((RB, HH), lambda i: (i, 0)),
            pl.BlockSpec((RB, 1), lambda i: (i, 0)),
            pl.BlockSpec((1, 1), lambda i: (0, 0)),
        ],
        out_specs=[
            pl.BlockSpec((2, RB, HC), lambda i: (0, i, 0)),
            pl.BlockSpec((RB, 1), lambda i: (i, 0)),
        ],
        out_shape=[
            jax.ShapeDtypeStruct((2, N, HC), jnp.float32),
            jax.ShapeDtypeStruct((N, 1), jnp.float32),
        ],
    )(xw, a, m)


def _sc_segment(ycat, wvec, src, dst, zero):
    E = src.shape[0]
    nch = E // (NS * K)
    assert nch * NS * K == E
    stripe = NPAD // NS
    mesh = plsc.VectorSubcoreMesh(core_axis_name="c", subcore_axis_name="s")

    @functools.partial(
        pl.kernel,
        mesh=mesh,
        out_type=[
            jax.ShapeDtypeStruct((2 * NPAD, HC), jnp.float32),
            jax.ShapeDtypeStruct((NC * NS * NL,), jnp.float32),
        ],
        scratch_types=[
            pltpu.VMEM((K,), jnp.int32),
            pltpu.VMEM((K,), jnp.int32),
            pltpu.VMEM((K,), jnp.int32),
            pltpu.VMEM((K, HC), jnp.float32),
            pltpu.VMEM((K,), jnp.float32),
            pltpu.VMEM((NL,), jnp.float32),
            pltpu.VMEM_SHARED((NPAD, HC), jnp.float32),
            pltpu.SemaphoreType.DMA,
            pltpu.SemaphoreType.DMA,
        ],
    )
    def k(ycat_hbm, w_hbm, src_hbm, dst_hbm, zero_hbm, s_hbm, z_hbm,
          idx_v, idxa_v, dst_v, rows_v, wch_v, zacc_v, acc_sh, sem1, sem2):
        c = lax.axis_index("c")
        s = lax.axis_index("s")
        wid = c * NS + s
        roff = c * N
        woff = c * NPAD
        pltpu.sync_copy(zero_hbm.at[pl.ds(s * stripe, stripe)],
                        acc_sh.at[pl.ds(s * stripe, stripe)])
        zacc_v[...] = jnp.zeros((NL,), jnp.float32)
        plsc.subcore_barrier()
        ebase = s * (E // NS)

        def chunk(j, carry):
            e0 = ebase + j * K
            pltpu.sync_copy(src_hbm.at[pl.ds(e0, K)], idx_v)
            pltpu.sync_copy(dst_hbm.at[pl.ds(e0, K)], dst_v)
            for t in range(K // NL):
                idxa_v[pl.ds(NL * t, NL)] = idx_v[pl.ds(NL * t, NL)] + roff
            g1 = pltpu.async_copy(ycat_hbm.at[idxa_v], rows_v, sem1)
            g2 = pltpu.async_copy(w_hbm.at[idx_v], wch_v, sem2)
            g1.wait()
            g2.wait()
            pltpu.sync_copy(rows_v, acc_sh.at[dst_v], add=True)
            zv = zacc_v[...]
            for t in range(K // NL):
                zv = zv + wch_v[pl.ds(NL * t, NL)]
            zacc_v[...] = zv
            return carry

        lax.fori_loop(0, nch, chunk, 0)
        plsc.subcore_barrier()
        pltpu.sync_copy(acc_sh.at[pl.ds(s * stripe, stripe)],
                        s_hbm.at[pl.ds(woff + s * stripe, stripe)])
        pltpu.sync_copy(zacc_v, z_hbm.at[pl.ds(wid * NL, NL)])

    return k(ycat, wvec, src, dst, zero)


def _update_body(x_ref, s0_ref, s1_ref, z_ref, wu_ref, bu_ref, o_ref):
    zinv = 2.0 / jnp.sum(z_ref[...])
    x = x_ref[...]
    acc = jnp.dot(x, wu_ref[0:D, :], preferred_element_type=jnp.float32)
    msum = jnp.dot(s0_ref[...], wu_ref[D:D + HC, :],
                   preferred_element_type=jnp.float32)
    msum += jnp.dot(s1_ref[...], wu_ref[D + HC:, :],
                    preferred_element_type=jnp.float32)
    o_ref[...] = jax.nn.relu(x + acc + msum * zinv + bu_ref[...])


def _node_update(x, s, z, wu, bu):
    s0 = lax.slice(s, (0, 0), (N, HC))
    s1 = lax.slice(s, (NPAD, 0), (NPAD + N, HC))
    nb = N // RB
    return pl.pallas_call(
        _update_body,
        grid=(nb,),
        in_specs=[
            pl.BlockSpec((RB, D), lambda i: (i, 0)),
            pl.BlockSpec((RB, HC), lambda i: (i, 0)),
            pl.BlockSpec((RB, HC), lambda i: (i, 0)),
            pl.BlockSpec((1, NC * NS * NL), lambda i: (0, 0)),
            pl.BlockSpec((2 * D, HH), lambda i: (0, 0)),
            pl.BlockSpec((1, HH), lambda i: (0, 0)),
        ],
        out_specs=pl.BlockSpec((RB, D), lambda i: (i, 0)),
        out_shape=jax.ShapeDtypeStruct((N, D), jnp.float32),
    )(x, s0, s1, z.reshape(1, -1), wu, bu.reshape(1, HH))


def kernel(place_features, transition_features, pre_edge_index, post_edge_index,
           W_ptm, b_ptm, W_tpm, b_tpm, W_pu, b_pu, W_tu, b_tu,
           W_pa, b_pa, W_ta, b_ta):
    pre = pre_edge_index.astype(jnp.int32)
    post = post_edge_index.astype(jnp.int32)
    zero = jnp.zeros((NPAD, HC), jnp.float32)

    xw_p, a_p, m_p = _node_transform(place_features, W_ptm, b_ptm, W_ta, b_ta)
    y_p, w_p = _node_weight(xw_p, a_p, m_p)
    s_p, z_p = _sc_segment(y_p.reshape(2 * N, HC), w_p.reshape(N),
                           pre[0], pre[1], zero)

    xw_t, a_t, m_t = _node_transform(transition_features, W_tpm, b_tpm,
                                     W_pa, b_pa)
    y_t, w_t = _node_weight(xw_t, a_t, m_t)
    s_t, z_t = _sc_segment(y_t.reshape(2 * N, HC), w_t.reshape(N),
                           post[0], post[1], zero)

    trans_out = _node_update(transition_features, s_p, z_p, W_tu, b_tu)
    place_out = _node_update(place_features, s_t, z_t, W_pu, b_pu)
    return (place_out, trans_out)


# speedup vs baseline: 6.2061x; 2.0196x over previous
import functools

import jax
import jax.numpy as jnp
from jax import lax
from jax.experimental import pallas as pl
from jax.experimental.pallas import tpu as pltpu
from jax.experimental.pallas import tpu_sc as plsc

N = 10000
D = 256
HH = 256
HC = 128
RB = 2000
K = 80
NC = 2
NS = 16
NL = 16
NPAD = 10240
PAKM = 16384


def _transform_body(x_ref, w_ref, b_ref, wa_ref, ba_ref,
                    xw_ref, a_ref, m_ref, msc):
    i = pl.program_id(0)
    xw = jnp.dot(x_ref[...], w_ref[...],
                 preferred_element_type=jnp.float32) + b_ref[...]
    xw_ref[...] = xw
    a = jnp.dot(xw, wa_ref[...],
                preferred_element_type=jnp.float32) + ba_ref[...]
    a_ref[...] = a
    bm = jnp.max(a)

    @pl.when(i == 0)
    def _():
        msc[0, 0] = bm

    @pl.when(i > 0)
    def _():
        msc[0, 0] = jnp.maximum(msc[0, 0], bm)

    m_ref[...] = jnp.full((1, 1), msc[0, 0], jnp.float32)


def _node_transform(x, w, b, wa, ba):
    return pl.pallas_call(
        _transform_body,
        grid=(N // RB,),
        in_specs=[
            pl.BlockSpec((RB, D), lambda i: (i, 0)),
            pl.BlockSpec((D, HH), lambda i: (0, 0)),
            pl.BlockSpec((1, HH), lambda i: (0, 0)),
            pl.BlockSpec((D, 1), lambda i: (0, 0)),
            pl.BlockSpec((1, 1), lambda i: (0, 0)),
        ],
        out_specs=[
            pl.BlockSpec((RB, HH), lambda i: (i, 0)),
            pl.BlockSpec((RB, 1), lambda i: (i, 0)),
            pl.BlockSpec((1, 1), lambda i: (0, 0)),
        ],
        out_shape=[
            jax.ShapeDtypeStruct((N, HH), jnp.float32),
            jax.ShapeDtypeStruct((N, 1), jnp.float32),
            jax.ShapeDtypeStruct((1, 1), jnp.float32),
        ],
        scratch_shapes=[pltpu.SMEM((1, 1), jnp.float32)],
    )(x, w, b.reshape(1, HH), wa, ba.reshape(1, 1))


def _weight_body(xw_ref, a_ref, m_ref, y_ref, w_ref):
    w = jnp.exp(a_ref[...] - m_ref[0, 0])
    w_ref[0, :, :] = w
    w_ref[1, :, :] = w
    y = xw_ref[...] * w
    y_ref[0, :, :] = y[:, :HC]
    y_ref[1, :, :] = y[:, HC:]


def _node_weight(xw, a, m):
    return pl.pallas_call(
        _weight_body,
        grid=(N // RB,),
        in_specs=[
            pl.BlockSpec((RB, HH), lambda i: (i, 0)),
            pl.BlockSpec((RB, 1), lambda i: (i, 0)),
            pl.BlockSpec((1, 1), lambda i: (0, 0)),
        ],
        out_specs=[
            pl.BlockSpec((2, RB, HC), lambda i: (0, i, 0)),
            pl.BlockSpec((2, RB, 1), lambda i: (0, i, 0)),
        ],
        out_shape=[
            jax.ShapeDtypeStruct((2, N, HC), jnp.float32),
            jax.ShapeDtypeStruct((2, N, 1), jnp.float32),
        ],
    )(xw, a, m)


def _sc_segment(ycat, wcat, pak3, zero):
    nch = pak3.shape[1]
    assert pak3.shape == (NS, nch, K) and nch % 2 == 1
    half = (nch - 1) // 2
    stripe = NPAD // NS
    mesh = plsc.VectorSubcoreMesh(core_axis_name="c", subcore_axis_name="s")

    @functools.partial(
        pl.kernel,
        mesh=mesh,
        out_type=[
            jax.ShapeDtypeStruct((2 * NPAD, HC), jnp.float32),
            jax.ShapeDtypeStruct((NC * NS * NL,), jnp.float32),
        ],
        scratch_types=[
            pltpu.VMEM((nch, K), jnp.int32),
            pltpu.VMEM((K,), jnp.int32),
            pltpu.VMEM((K,), jnp.int32),
            pltpu.VMEM((K,), jnp.int32),
            pltpu.VMEM((K,), jnp.int32),
            pltpu.VMEM((K, HC), jnp.float32),
            pltpu.VMEM((K, HC), jnp.float32),
            pltpu.VMEM((K,), jnp.float32),
            pltpu.VMEM((K,), jnp.float32),
            pltpu.VMEM((NL,), jnp.float32),
            pltpu.VMEM_SHARED((NPAD, HC), jnp.float32),
            pltpu.SemaphoreType.DMA,
            pltpu.SemaphoreType.DMA,
            pltpu.SemaphoreType.DMA,
            pltpu.SemaphoreType.DMA,
        ],
    )
    def k(ycat_hbm, w_hbm, pak_hbm, zero_hbm, s_hbm, z_hbm,
          pakA, idxa0, idxa1, dstb0, dstb1, rows0, rows1, wch0, wch1,
          zacc_v, acc_sh, semr0, semr1, semw0, semw1):
        c = lax.axis_index("c")
        s = lax.axis_index("s")
        wid = c * NS + s
        roff = c * N
        woff = c * NPAD
        pltpu.sync_copy(zero_hbm.at[pl.ds(s * stripe, stripe)],
                        acc_sh.at[pl.ds(s * stripe, stripe)])
        pltpu.sync_copy(pak_hbm.at[s], pakA)
        zacc_v[...] = jnp.zeros((NL,), jnp.float32)
        plsc.subcore_barrier()

        bufs = ((idxa0, dstb0, rows0, wch0, semr0, semw0),
                (idxa1, dstb1, rows1, wch1, semr1, semw1))

        def fire(j, p):
            idxa, dstb, rows, wch, semr, semw = bufs[p]
            for t in range(K // NL):
                v = pakA[j, pl.ds(NL * t, NL)]
                dstb[pl.ds(NL * t, NL)] = lax.shift_right_logical(v, 14)
                idxa[pl.ds(NL * t, NL)] = (v & (PAKM - 1)) + roff
            pltpu.async_copy(ycat_hbm.at[idxa], rows, semr)
            pltpu.async_copy(w_hbm.at[idxa], wch, semw)

        def consume(j, p):
            idxa, dstb, rows, wch, semr, semw = bufs[p]
            pltpu.make_async_copy(ycat_hbm.at[idxa], rows, semr).wait()
            pltpu.make_async_copy(w_hbm.at[idxa], wch, semw).wait()
            pltpu.sync_copy(rows, acc_sh.at[dstb], add=True)
            zv = zacc_v[...]
            for t in range(K // NL):
                zv = zv + wch[pl.ds(NL * t, NL)]
            zacc_v[...] = zv

        fire(0, 0)

        def body(i, carry):
            fire(2 * i + 1, 1)
            consume(2 * i, 0)
            fire(2 * i + 2, 0)
            consume(2 * i + 1, 1)
            return carry

        lax.fori_loop(0, half, body, 0)
        consume(nch - 1, 0)
        plsc.subcore_barrier()
        pltpu.sync_copy(acc_sh.at[pl.ds(s * stripe, stripe)],
                        s_hbm.at[pl.ds(woff + s * stripe, stripe)])
        pltpu.sync_copy(zacc_v, z_hbm.at[pl.ds(wid * NL, NL)])

    return k(ycat, wcat, pak3, zero)


def _update_body(x_ref, s0_ref, s1_ref, z_ref, wu_ref, bu_ref, o_ref):
    zinv = 2.0 / jnp.sum(z_ref[...])
    x = x_ref[...]
    acc = jnp.dot(x, wu_ref[0:D, :], preferred_element_type=jnp.float32)
    msum = jnp.dot(s0_ref[...], wu_ref[D:D + HC, :],
                   preferred_element_type=jnp.float32)
    msum += jnp.dot(s1_ref[...], wu_ref[D + HC:, :],
                    preferred_element_type=jnp.float32)
    o_ref[...] = jax.nn.relu(x + acc + msum * zinv + bu_ref[...])


def _node_update(x, s, z, wu, bu):
    s0 = lax.slice(s, (0, 0), (N, HC))
    s1 = lax.slice(s, (NPAD, 0), (NPAD + N, HC))
    nb = N // RB
    return pl.pallas_call(
        _update_body,
        grid=(nb,),
        in_specs=[
            pl.BlockSpec((RB, D), lambda i: (i, 0)),
            pl.BlockSpec((RB, HC), lambda i: (i, 0)),
            pl.BlockSpec((RB, HC), lambda i: (i, 0)),
            pl.BlockSpec((1, NC * NS * NL), lambda i: (0, 0)),
            pl.BlockSpec((2 * D, HH), lambda i: (0, 0)),
            pl.BlockSpec((1, HH), lambda i: (0, 0)),
        ],
        out_specs=pl.BlockSpec((RB, D), lambda i: (i, 0)),
        out_shape=jax.ShapeDtypeStruct((N, D), jnp.float32),
    )(x, s0, s1, z.reshape(1, -1), wu, bu.reshape(1, HH))


def kernel(place_features, transition_features, pre_edge_index, post_edge_index,
           W_ptm, b_ptm, W_tpm, b_tpm, W_pu, b_pu, W_tu, b_tu,
           W_pa, b_pa, W_ta, b_ta):
    E = pre_edge_index.shape[1]
    nch = E // (NS * K)
    pre = pre_edge_index.astype(jnp.int32)
    post = post_edge_index.astype(jnp.int32)
    pak_pre = (pre[0] + pre[1] * PAKM).reshape(NS, nch, K)
    pak_post = (post[0] + post[1] * PAKM).reshape(NS, nch, K)
    zero = jnp.zeros((NPAD, HC), jnp.float32)

    xw_p, a_p, m_p = _node_transform(place_features, W_ptm, b_ptm, W_ta, b_ta)
    y_p, w_p = _node_weight(xw_p, a_p, m_p)
    s_p, z_p = _sc_segment(y_p.reshape(2 * N, HC), w_p.reshape(2 * N),
                           pak_pre, zero)

    xw_t, a_t, m_t = _node_transform(transition_features, W_tpm, b_tpm,
                                     W_pa, b_pa)
    y_t, w_t = _node_weight(xw_t, a_t, m_t)
    s_t, z_t = _sc_segment(y_t.reshape(2 * N, HC), w_t.reshape(2 * N),
                           pak_post, zero)

    trans_out = _node_update(transition_features, s_p, z_p, W_tu, b_tu)
    place_out = _node_update(place_features, s_t, z_t, W_pu, b_pu)
    return (place_out, trans_out)
